# baseline (speedup 1.0000x reference)
import functools

import jax
import jax.numpy as jnp
from jax import lax
from jax.experimental import pallas as pl
from jax.experimental.pallas import tpu as pltpu
from jax.experimental.pallas import tpu_sc as plsc

D_MODEL = 1024
NUM_CORES = 2
NUM_SUBCORES = 16
NUM_WORKERS = NUM_CORES * NUM_SUBCORES
LANES = 16
CHUNK = 8
DEPTH = 4


def _sc_body(x_hbm, pos_hbm, table_hbm, out_hbm, idx_v,
             bx0, bx1, bx2, bx3, bt0, bt1, bt2, bt3, bo0, bo1, bo2, bo3,
             sg0, sg1, sg2, sg3, sx0, sx1, sx2, sx3, so0, so1, so2, so3):
    bx = [bx0, bx1, bx2, bx3]
    bt = [bt0, bt1, bt2, bt3]
    bo = [bo0, bo1, bo2, bo3]
    sg = [sg0, sg1, sg2, sg3]
    sx = [sx0, sx1, sx2, sx3]
    so = [so0, so1, so2, so3]

    wid = lax.axis_index("s") * NUM_CORES + lax.axis_index("c")
    n_per_w = x_hbm.shape[0] // NUM_WORKERS
    base_w = wid * n_per_w
    steps = n_per_w // CHUNK
    pltpu.sync_copy(pos_hbm.at[pl.ds(base_w, n_per_w)], idx_v)

    def issue(st, b):
        base = base_w + st * CHUNK
        pltpu.async_copy(
            table_hbm.at[idx_v.at[pl.ds(st * CHUNK, CHUNK)]], bt[b], sg[b])
        pltpu.async_copy(x_hbm.at[pl.ds(base, CHUNK)], bx[b], sx[b])

    def wait_in(b):
        pltpu.make_async_copy(
            table_hbm.at[idx_v.at[pl.ds(0, CHUNK)]], bt[b], sg[b]).wait()
        pltpu.make_async_copy(
            x_hbm.at[pl.ds(base_w, CHUNK)], bx[b], sx[b]).wait()

    def wait_out(b):
        pltpu.make_async_copy(
            bo[b], out_hbm.at[pl.ds(base_w, CHUNK)], so[b]).wait()

    def add_half(b, h):
        def row_fn(r, c):
            for j in range(D_MODEL // LANES):
                sl = pl.ds(j * LANES, LANES)
                bo[b][r, sl] = bx[b][r, sl] + bt[b][r, sl]
            return c

        lax.fori_loop(h * (CHUNK // 2), (h + 1) * (CHUNK // 2), row_fn, 0)

    def store_half(st, b, h):
        base = base_w + st * CHUNK + h * (CHUNK // 2)
        pltpu.async_copy(
            bo[b].at[pl.ds(h * (CHUNK // 2), CHUNK // 2)],
            out_hbm.at[pl.ds(base, CHUNK // 2)], so[b])

    for b in range(DEPTH):
        issue(b, b)

    def group_fn(i, c):
        st0 = i * DEPTH
        for b in range(DEPTH):
            st = st0 + b
            wait_in(b)

            @pl.when(i > 0)
            def _():
                wait_out(b)

            add_half(b, 0)
            store_half(st, b, 0)
            add_half(b, 1)

            @pl.when(st + DEPTH < steps)
            def _():
                issue(st + DEPTH, b)

            store_half(st, b, 1)
        return c

    lax.fori_loop(0, steps // DEPTH, group_fn, 0)

    for b in range(DEPTH):
        wait_out(b)


def _build(n_tokens):
    return functools.partial(
        pl.kernel,
        out_type=jax.ShapeDtypeStruct((n_tokens, D_MODEL), jnp.float32),
        mesh=plsc.VectorSubcoreMesh(
            core_axis_name="c",
            subcore_axis_name="s",
            num_cores=NUM_CORES,
            num_subcores=NUM_SUBCORES,
        ),
        scratch_types=(
            [pltpu.VMEM((n_tokens // NUM_WORKERS,), jnp.int32)]
            + [pltpu.VMEM((CHUNK, D_MODEL), jnp.float32)
               for _ in range(3 * DEPTH)]
            + [pltpu.SemaphoreType.DMA for _ in range(3 * DEPTH)]
        ),
    )(_sc_body)


@jax.jit
def _run(x_flat, pos_flat, pos_table):
    return _build(x_flat.shape[0])(x_flat, pos_flat, pos_table)


def kernel(x, positions, pos_table):
    b, s, d = x.shape
    x_flat = x.reshape(b * s, d)
    pos_flat = positions.reshape(b * s).astype(jnp.int32)
    out = _run(x_flat, pos_flat, pos_table)
    return out.reshape(b, s, d)


# speedup vs baseline: 1.1879x; 1.1879x over previous
import functools

import jax
import jax.numpy as jnp
from jax import lax
from jax.experimental import pallas as pl
from jax.experimental.pallas import tpu as pltpu
from jax.experimental.pallas import tpu_sc as plsc

D_MODEL = 1024
NUM_CORES = 2
NUM_SUBCORES = 16
NUM_WORKERS = NUM_CORES * NUM_SUBCORES
LANES = 16
CHUNK = 8
DEPTH = 4


def _sc_body(x_hbm, pos_hbm, table_hbm, out_hbm, idx_v,
             bx0, bx1, bx2, bx3, bt0, bt1, bt2, bt3, bo0, bo1, bo2, bo3,
             sg0, sg1, sg2, sg3, sx0, sx1, sx2, sx3, so0, so1, so2, so3):
    bx = [bx0, bx1, bx2, bx3]
    bt = [bt0, bt1, bt2, bt3]
    bo = [bo0, bo1, bo2, bo3]
    sg = [sg0, sg1, sg2, sg3]
    sx = [sx0, sx1, sx2, sx3]
    so = [so0, so1, so2, so3]

    wid = lax.axis_index("s") * NUM_CORES + lax.axis_index("c")
    n_per_w = x_hbm.shape[0] // NUM_WORKERS
    base_w = wid * n_per_w
    steps = n_per_w // CHUNK
    pltpu.sync_copy(pos_hbm.at[pl.ds(base_w, n_per_w)], idx_v)

    def issue(st, b):
        base = base_w + st * CHUNK
        pltpu.async_copy(
            table_hbm.at[idx_v.at[pl.ds(st * CHUNK, CHUNK)]], bt[b], sg[b])
        pltpu.async_copy(x_hbm.at[pl.ds(base, CHUNK)], bx[b], sx[b])

    def wait_in(b):
        pltpu.make_async_copy(
            table_hbm.at[idx_v.at[pl.ds(0, CHUNK)]], bt[b], sg[b]).wait()
        pltpu.make_async_copy(
            x_hbm.at[pl.ds(base_w, CHUNK)], bx[b], sx[b]).wait()

    def wait_out(b):
        pltpu.make_async_copy(
            bo[b], out_hbm.at[pl.ds(base_w, CHUNK)], so[b]).wait()

    def add(b):
        def row_fn(r, c):
            for j in range(D_MODEL // LANES):
                sl = pl.ds(j * LANES, LANES)
                bo[b][r, sl] = bx[b][r, sl] + bt[b][r, sl]
            return c

        lax.fori_loop(0, CHUNK, row_fn, 0)

    def store(st, b):
        base = base_w + st * CHUNK
        pltpu.async_copy(bo[b], out_hbm.at[pl.ds(base, CHUNK)], so[b])

    for b in range(DEPTH):
        issue(b, b)

    def group_fn(i, c):
        st0 = i * DEPTH
        for b in range(DEPTH):
            st = st0 + b
            wait_in(b)

            @pl.when(i > 0)
            def _():
                wait_out(b)

            add(b)

            @pl.when(st + DEPTH < steps)
            def _():
                issue(st + DEPTH, b)

            store(st, b)
        return c

    lax.fori_loop(0, steps // DEPTH, group_fn, 0)

    for b in range(DEPTH):
        wait_out(b)


def _build(n_tokens):
    return functools.partial(
        pl.kernel,
        out_type=jax.ShapeDtypeStruct((n_tokens, D_MODEL), jnp.float32),
        mesh=plsc.VectorSubcoreMesh(
            core_axis_name="c",
            subcore_axis_name="s",
            num_cores=NUM_CORES,
            num_subcores=NUM_SUBCORES,
        ),
        scratch_types=(
            [pltpu.VMEM((n_tokens // NUM_WORKERS,), jnp.int32)]
            + [pltpu.VMEM((CHUNK, D_MODEL), jnp.float32)
               for _ in range(3 * DEPTH)]
            + [pltpu.SemaphoreType.DMA for _ in range(3 * DEPTH)]
        ),
    )(_sc_body)


@jax.jit
def _run(x_flat, pos_flat, pos_table):
    return _build(x_flat.shape[0])(x_flat, pos_flat, pos_table)


def kernel(x, positions, pos_table):
    b, s, d = x.shape
    x_flat = x.reshape(b * s, d)
    pos_flat = positions.reshape(b * s).astype(jnp.int32)
    out = _run(x_flat, pos_flat, pos_table)
    return out.reshape(b, s, d)
